# baseline (speedup 1.0000x reference)
import functools

import jax
import jax.numpy as jnp
from jax import lax
from jax.experimental import pallas as pl
from jax.experimental.pallas import tpu as pltpu
from jax.experimental.pallas import tpu_sc as plsc

B, N1, N2, C1, C2 = 8, 4096, 1024, 128, 256
BN = B * N1
TILE = 512
TILE3 = 1024
NW = 32
PW = BN // NW
CH = 64


def _topk_body(x1t_ref, x2_ref, idx_ref, w_ref):
    b = pl.program_id(0)
    x1t = x1t_ref[0]
    x2 = x2_ref[0]
    x2sq = jnp.sum(x2 * x2, axis=1, keepdims=True)
    key = x2sq + jnp.dot(x2, x1t * -2.0,
                         precision=lax.Precision.HIGHEST,
                         preferred_element_type=jnp.float32)
    x1sq = jnp.sum(x1t * x1t, axis=0, keepdims=True)
    iota = lax.broadcasted_iota(jnp.int32, (N2, TILE), 0)
    cur = key
    ims, ms = [], []
    for k in range(3):
        m = jnp.min(cur, axis=0, keepdims=True)
        hit = cur == m
        im = jnp.min(jnp.where(hit, iota, N2), axis=0, keepdims=True)
        ms.append(m)
        ims.append(im)
        if k < 2:
            cur = jnp.where(hit, jnp.float32(jnp.inf), cur)
    dist3 = jnp.maximum(jnp.concatenate(ms, axis=0) + x1sq, 1e-10)
    inv = 1.0 / dist3
    w3 = inv / jnp.sum(inv, axis=0, keepdims=True)
    idx3 = jnp.concatenate(ims, axis=0) + b * N2
    idx_ref[0] = idx3
    w_ref[0] = w3


def _topk_stage(xyz1t, xyz2):
    return pl.pallas_call(
        _topk_body,
        grid=(B, N1 // TILE),
        in_specs=[
            pl.BlockSpec((1, 3, TILE), lambda b, t: (b, 0, t)),
            pl.BlockSpec((1, N2, 3), lambda b, t: (b, 0, 0)),
        ],
        out_specs=[
            pl.BlockSpec((1, 3, TILE), lambda b, t: (b, 0, t)),
            pl.BlockSpec((1, 3, TILE), lambda b, t: (b, 0, t)),
        ],
        out_shape=[
            jax.ShapeDtypeStruct((B, 3, N1), jnp.int32),
            jax.ShapeDtypeStruct((B, 3, N1), jnp.float32),
        ],
    )(xyz1t, xyz2)


def _sc_interp(idx_pl, w_pl, table):
    mesh = plsc.VectorSubcoreMesh(core_axis_name="c", subcore_axis_name="s")

    @functools.partial(
        pl.kernel,
        mesh=mesh,
        compiler_params=pltpu.CompilerParams(use_tc_tiling_on_sc=False,
                                             needs_layout_passes=False),
        out_type=jax.ShapeDtypeStruct((BN, C2), jnp.float32),
        scratch_types=[
            pltpu.VMEM((3, PW), jnp.int32),
            pltpu.VMEM((3, PW + 16), jnp.float32),
            pltpu.VMEM((CH, C2), jnp.float32),
            pltpu.VMEM((CH, C2), jnp.float32),
            pltpu.VMEM((CH, C2), jnp.float32),
            pltpu.VMEM((CH, C2), jnp.float32),
            pltpu.SemaphoreType.DMA,
        ],
    )
    def k(idx_hbm, w_hbm, table_hbm, out_hbm,
          iv, wv, r0, r1, r2, ov, sem):
        wid = lax.axis_index("s") * 2 + lax.axis_index("c")
        base_w = wid * PW
        bb = wid // (N1 // PW)
        rr = (wid % (N1 // PW)) * PW
        zero16 = jnp.zeros((16, 1), jnp.int32)
        dnums = lax.GatherDimensionNumbers(
            offset_dims=(), collapsed_slice_dims=(0,), start_index_map=(0,))

        def bcast0(vec):
            return lax.gather(vec, zero16, dnums, slice_sizes=(1,),
                              mode=lax.GatherScatterMode.PROMISE_IN_BOUNDS)
        pltpu.sync_copy(idx_hbm.at[bb, :, pl.ds(rr, PW)], iv)
        pltpu.sync_copy(w_hbm.at[bb, :, pl.ds(rr, PW)],
                        wv.at[:, pl.ds(0, PW)])

        def chunk(ci, _):
            off = ci * CH
            cp0 = pltpu.async_copy(table_hbm.at[iv.at[0, pl.ds(off, CH)]], r0, sem)
            cp1 = pltpu.async_copy(table_hbm.at[iv.at[1, pl.ds(off, CH)]], r1, sem)
            cp2 = pltpu.async_copy(table_hbm.at[iv.at[2, pl.ds(off, CH)]], r2, sem)
            cp0.wait()
            cp1.wait()
            cp2.wait()

            @plsc.parallel_loop(0, CH, unroll=4)
            def point(p):
                b0 = bcast0(wv[0, pl.ds(off + p, 16)])
                b1 = bcast0(wv[1, pl.ds(off + p, 16)])
                b2 = bcast0(wv[2, pl.ds(off + p, 16)])
                for c in range(C2 // 16):
                    s = pl.ds(c * 16, 16)
                    t01 = r0[p, s] * b0 + r1[p, s] * b1
                    ov[p, s] = t01 + r2[p, s] * b2

            pltpu.sync_copy(ov, out_hbm.at[pl.ds(base_w + off, CH)])
            return 0

        lax.fori_loop(0, PW // CH, chunk, 0)

    return k(idx_pl, w_pl, table)


def _mlp_body(interp_ref, f1_ref, w1a_ref, w1b_ref, b1_ref, w2_ref, b2_ref, out_ref):
    h = jnp.dot(interp_ref[...], w1a_ref[...], preferred_element_type=jnp.float32)
    h = h + jnp.dot(f1_ref[...], w1b_ref[...], preferred_element_type=jnp.float32)
    h = jnp.maximum(h + b1_ref[...], 0.0)
    o = jnp.dot(h, w2_ref[...], preferred_element_type=jnp.float32)
    out_ref[...] = jnp.maximum(o + b2_ref[...], 0.0)


def _mlp_stage(interp, f1, w1a, w1b, b1, w2, b2):
    return pl.pallas_call(
        _mlp_body,
        grid=(BN // TILE3,),
        in_specs=[
            pl.BlockSpec((TILE3, C2), lambda t: (t, 0)),
            pl.BlockSpec((TILE3, C1), lambda t: (t, 0)),
            pl.BlockSpec((C2, 256), lambda t: (0, 0)),
            pl.BlockSpec((C1, 256), lambda t: (0, 0)),
            pl.BlockSpec((1, 256), lambda t: (0, 0)),
            pl.BlockSpec((256, 256), lambda t: (0, 0)),
            pl.BlockSpec((1, 256), lambda t: (0, 0)),
        ],
        out_specs=pl.BlockSpec((TILE3, 256), lambda t: (t, 0)),
        out_shape=jax.ShapeDtypeStruct((BN, 256), jnp.float32),
    )(interp, f1, w1a, w1b, b1, w2, b2)


def kernel(xyz1, xyz2, features1, features2, W1, b1, W2, b2):
    xyz1t = jnp.transpose(xyz1, (0, 2, 1))
    idx3, w3 = _topk_stage(xyz1t, xyz2)
    table = features2.reshape(B * N2, C2)
    interp = _sc_interp(idx3, w3, table)
    out = _mlp_stage(interp, features1.reshape(BN, C1),
                     W1[:C2], W1[C2:], b1.reshape(1, 256),
                     W2, b2.reshape(1, 256))
    return out.reshape(B, N1, 256)


# speedup vs baseline: 1.3416x; 1.3416x over previous
import functools

import jax
import jax.numpy as jnp
from jax import lax
from jax.experimental import pallas as pl
from jax.experimental.pallas import tpu as pltpu
from jax.experimental.pallas import tpu_sc as plsc

B, N1, N2, C1, C2 = 8, 4096, 1024, 128, 256
BN = B * N1
TILE = 512
TILE3 = 1024
NW = 32
PW = BN // NW
CH = 64


def _topk_body(x1t_ref, x2_ref, idx_ref, w_ref):
    b = pl.program_id(0)
    x1t = x1t_ref[0]
    x2 = x2_ref[0]
    d = None
    for c in range(3):
        term = x2[:, c:c + 1] - x1t[c:c + 1, :]
        sq = term * term
        d = sq if d is None else d + sq
    iota = lax.broadcasted_iota(jnp.int32, (N2, TILE), 0).astype(jnp.float32)
    cur = d
    ims, ms = [], []
    for k in range(3):
        m = jnp.min(cur, axis=0, keepdims=True)
        hit = cur == m
        im = jnp.min(jnp.where(hit, iota, jnp.float32(N2)),
                     axis=0, keepdims=True)
        ms.append(m)
        ims.append(im)
        if k < 2:
            cur = jnp.where(hit, jnp.float32(jnp.inf), cur)
    dist3 = jnp.maximum(jnp.concatenate(ms, axis=0), 1e-10)
    inv = 1.0 / dist3
    w3 = inv / jnp.sum(inv, axis=0, keepdims=True)
    idx3 = jnp.concatenate(ims, axis=0).astype(jnp.int32) + b * N2
    idx_ref[0] = idx3
    w_ref[0] = w3


def _topk_stage(xyz1t, xyz2):
    nb = xyz1t.shape[0]
    return pl.pallas_call(
        _topk_body,
        grid=(nb, N1 // TILE),
        in_specs=[
            pl.BlockSpec((1, 3, TILE), lambda b, t: (b, 0, t)),
            pl.BlockSpec((1, N2, 3), lambda b, t: (b, 0, 0)),
        ],
        out_specs=[
            pl.BlockSpec((1, 3, TILE), lambda b, t: (b, 0, t)),
            pl.BlockSpec((1, 3, TILE), lambda b, t: (b, 0, t)),
        ],
        out_shape=[
            jax.ShapeDtypeStruct((nb, 3, N1), jnp.int32),
            jax.ShapeDtypeStruct((nb, 3, N1), jnp.float32),
        ],
    )(xyz1t, xyz2)


def _sc_interp(idx_pl, w_pl, table):
    mesh = plsc.VectorSubcoreMesh(core_axis_name="c", subcore_axis_name="s")
    nb = idx_pl.shape[0]
    npts = nb * N1
    pw = npts // NW
    wpb = N1 // pw

    @functools.partial(
        pl.kernel,
        mesh=mesh,
        compiler_params=pltpu.CompilerParams(use_tc_tiling_on_sc=False,
                                             needs_layout_passes=False),
        out_type=jax.ShapeDtypeStruct((npts, C2), jnp.float32),
        scratch_types=[
            pltpu.VMEM((3, pw), jnp.int32),
            pltpu.VMEM((3, pw + 16), jnp.float32),
            pltpu.VMEM((CH, C2), jnp.float32),
            pltpu.VMEM((CH, C2), jnp.float32),
            pltpu.VMEM((CH, C2), jnp.float32),
            pltpu.VMEM((CH, C2), jnp.float32),
            pltpu.SemaphoreType.DMA,
        ],
    )
    def k(idx_hbm, w_hbm, table_hbm, out_hbm,
          iv, wv, r0, r1, r2, ov, sem):
        wid = lax.axis_index("s") * 2 + lax.axis_index("c")
        base_w = wid * pw
        bb = wid // wpb
        rr = (wid % wpb) * pw
        zero16 = jnp.zeros((16, 1), jnp.int32)
        dnums = lax.GatherDimensionNumbers(
            offset_dims=(), collapsed_slice_dims=(0,), start_index_map=(0,))

        def bcast0(vec):
            return lax.gather(vec, zero16, dnums, slice_sizes=(1,),
                              mode=lax.GatherScatterMode.PROMISE_IN_BOUNDS)
        pltpu.sync_copy(idx_hbm.at[bb, :, pl.ds(rr, pw)], iv)
        pltpu.sync_copy(w_hbm.at[bb, :, pl.ds(rr, pw)],
                        wv.at[:, pl.ds(0, pw)])

        def chunk(ci, _):
            off = ci * CH
            cp0 = pltpu.async_copy(table_hbm.at[iv.at[0, pl.ds(off, CH)]], r0, sem)
            cp1 = pltpu.async_copy(table_hbm.at[iv.at[1, pl.ds(off, CH)]], r1, sem)
            cp2 = pltpu.async_copy(table_hbm.at[iv.at[2, pl.ds(off, CH)]], r2, sem)
            cp0.wait()
            cp1.wait()
            cp2.wait()

            @plsc.parallel_loop(0, CH, unroll=4)
            def point(p):
                b0 = bcast0(wv[0, pl.ds(off + p, 16)])
                b1 = bcast0(wv[1, pl.ds(off + p, 16)])
                b2 = bcast0(wv[2, pl.ds(off + p, 16)])
                for c in range(C2 // 16):
                    s = pl.ds(c * 16, 16)
                    t01 = r0[p, s] * b0 + r1[p, s] * b1
                    ov[p, s] = t01 + r2[p, s] * b2

            pltpu.sync_copy(ov, out_hbm.at[pl.ds(base_w + off, CH)])
            return 0

        lax.fori_loop(0, pw // CH, chunk, 0)

    return k(idx_pl, w_pl, table)


def _mlp_body(interp_ref, f1_ref, w1a_ref, w1b_ref, b1_ref, w2_ref, b2_ref, out_ref):
    h = jnp.dot(interp_ref[...], w1a_ref[...], preferred_element_type=jnp.float32)
    h = h + jnp.dot(f1_ref[...], w1b_ref[...], preferred_element_type=jnp.float32)
    h = jnp.maximum(h + b1_ref[...], 0.0)
    o = jnp.dot(h, w2_ref[...], preferred_element_type=jnp.float32)
    out_ref[...] = jnp.maximum(o + b2_ref[...], 0.0)


def _mlp_stage(interp, f1, w1a, w1b, b1, w2, b2):
    nrows = interp.shape[0]
    return pl.pallas_call(
        _mlp_body,
        grid=(nrows // TILE3,),
        in_specs=[
            pl.BlockSpec((TILE3, C2), lambda t: (t, 0)),
            pl.BlockSpec((TILE3, C1), lambda t: (t, 0)),
            pl.BlockSpec((C2, 256), lambda t: (0, 0)),
            pl.BlockSpec((C1, 256), lambda t: (0, 0)),
            pl.BlockSpec((1, 256), lambda t: (0, 0)),
            pl.BlockSpec((256, 256), lambda t: (0, 0)),
            pl.BlockSpec((1, 256), lambda t: (0, 0)),
        ],
        out_specs=pl.BlockSpec((TILE3, 256), lambda t: (t, 0)),
        out_shape=jax.ShapeDtypeStruct((nrows, 256), jnp.float32),
    )(interp, f1, w1a, w1b, b1, w2, b2)


NSLICE = 4


def kernel(xyz1, xyz2, features1, features2, W1, b1, W2, b2):
    xyz1t = jnp.transpose(xyz1, (0, 2, 1))
    w1a, w1b = W1[:C2], W1[C2:]
    b1r, b2r = b1.reshape(1, 256), b2.reshape(1, 256)
    bs = B // NSLICE
    outs = []
    for s in range(NSLICE):
        sl = slice(s * bs, (s + 1) * bs)
        idx3, w3 = _topk_stage(xyz1t[sl], xyz2[sl])
        table = features2[sl].reshape(bs * N2, C2)
        interp = _sc_interp(idx3, w3, table)
        outs.append(_mlp_stage(interp, features1[sl].reshape(bs * N1, C1),
                               w1a, w1b, b1r, W2, b2r))
    return jnp.concatenate(outs, axis=0).reshape(B, N1, 256)
